# baseline (speedup 1.0000x reference)
import jax
import jax.numpy as jnp
from jax.experimental import pallas as pl
from jax.experimental.pallas import tpu as pltpu

_ROWS = 1000000
_DIM = 64
_BC = 61440


def _copy_block(in_ref, out_ref):
    out_ref[...] = in_ref[...]


def kernel(weight):
    wt = weight.T
    out_t = pl.pallas_call(
        _copy_block,
        grid=(pl.cdiv(_ROWS, _BC),),
        in_specs=[pl.BlockSpec((_DIM, _BC), lambda i: (0, i))],
        out_specs=pl.BlockSpec((_DIM, _BC), lambda i: (0, i)),
        out_shape=jax.ShapeDtypeStruct((_DIM, _ROWS), jnp.float32),
        compiler_params=pltpu.CompilerParams(
            dimension_semantics=("arbitrary",),
            vmem_limit_bytes=64 * 1024 * 1024,
        ),
    )(wt)
    return out_t.T
